# baseline (speedup 1.0000x reference)
import jax
import jax.numpy as jnp
from jax.experimental import pallas as pl
from jax.experimental.pallas import tpu as pltpu

_TOP_K = 8
_BLOCK_T = 4096
_SPLIT_D = 2


def _gate_kernel(x_ref, w_ref, b_ref, wout_ref, iout_ref, acc_ref):
    dd = pl.program_id(1)
    partial = jax.lax.dot_general(
        w_ref[...], x_ref[...], (((1,), (1,)), ((), ())),
        preferred_element_type=jnp.float32,
    )

    @pl.when(dd == 0)
    def _first():
        acc_ref[...] = partial

    @pl.when(dd == _SPLIT_D - 1)
    def _select():
        logits = acc_ref[...] + partial
        sig = jax.nn.sigmoid(logits)
        work = logits + b_ref[...]
        n_exp = work.shape[0]
        iota_f = jax.lax.broadcasted_iota(jnp.int32, work.shape, 0).astype(
            jnp.float32)
        key = iota_f + 0.5 * sig
        neg_inf = jnp.float32(-jnp.inf)
        big = jnp.float32(n_exp)
        vs = []
        for _ in range(_TOP_K):
            m = jnp.max(work, axis=0, keepdims=True)
            v = jnp.min(jnp.where(work == m, key, big), axis=0, keepdims=True)
            vs.append(v)
            work = jnp.where(key == v, neg_inf, work)
        vmat = jnp.concatenate(vs, axis=0)
        idx_f = jnp.floor(vmat)
        wmat = 2.0 * (vmat - idx_f)
        wout_ref[...] = wmat / jnp.sum(wmat, axis=0, keepdims=True)
        iout_ref[...] = idx_f.astype(jnp.int32)


def kernel(x, W, expert_bias):
    t, d = x.shape
    e = W.shape[0]
    dsub = d // _SPLIT_D
    bias2d = expert_bias.reshape(e, 1)
    wout_t, iout_t = pl.pallas_call(
        _gate_kernel,
        grid=(t // _BLOCK_T, _SPLIT_D),
        in_specs=[
            pl.BlockSpec((_BLOCK_T, dsub), lambda i, dd: (i, dd)),
            pl.BlockSpec((e, dsub), lambda i, dd: (0, dd)),
            pl.BlockSpec((e, 1), lambda i, dd: (0, 0)),
        ],
        out_specs=[
            pl.BlockSpec((_TOP_K, _BLOCK_T), lambda i, dd: (0, i)),
            pl.BlockSpec((_TOP_K, _BLOCK_T), lambda i, dd: (0, i)),
        ],
        out_shape=[
            jax.ShapeDtypeStruct((_TOP_K, t), jnp.float32),
            jax.ShapeDtypeStruct((_TOP_K, t), jnp.int32),
        ],
        scratch_shapes=[pltpu.VMEM((e, _BLOCK_T), jnp.float32)],
    )(x, W, bias2d)
    return (wout_t.T, iout_t.T)


# speedup vs baseline: 1.5996x; 1.5996x over previous
import jax
import jax.numpy as jnp
from jax.experimental import pallas as pl

_TOP_K = 8
_BLOCK_T = 4096


def _gate_kernel(x_ref, w_ref, b_ref, wout_ref, iout_ref):
    x = x_ref[...]
    w = w_ref[...]
    logits = jax.lax.dot_general(
        w, x, (((1,), (1,)), ((), ())), preferred_element_type=jnp.float32
    )
    sig = jax.nn.sigmoid(logits)
    work = logits + b_ref[...]
    n_exp = work.shape[0]
    iota_f = jax.lax.broadcasted_iota(jnp.int32, work.shape, 0).astype(jnp.float32)
    key = iota_f + 0.5 * sig
    neg_inf = jnp.float32(-jnp.inf)
    big = jnp.float32(n_exp)
    vs = []
    for _ in range(_TOP_K):
        m = jnp.max(work, axis=0, keepdims=True)
        v = jnp.min(jnp.where(work == m, key, big), axis=0, keepdims=True)
        vs.append(v)
        work = jnp.where(key == v, neg_inf, work)
    vmat = jnp.concatenate(vs, axis=0)
    idx_f = jnp.floor(vmat)
    wmat = 2.0 * (vmat - idx_f)
    wout_ref[...] = wmat / jnp.sum(wmat, axis=0, keepdims=True)
    iout_ref[...] = idx_f.astype(jnp.int32)


def kernel(x, W, expert_bias):
    t, d = x.shape
    e = W.shape[0]
    bias2d = expert_bias.reshape(e, 1)
    wout_t, iout_t = pl.pallas_call(
        _gate_kernel,
        grid=(t // _BLOCK_T,),
        in_specs=[
            pl.BlockSpec((_BLOCK_T, d), lambda i: (i, 0)),
            pl.BlockSpec((e, d), lambda i: (0, 0)),
            pl.BlockSpec((e, 1), lambda i: (0, 0)),
        ],
        out_specs=[
            pl.BlockSpec((_TOP_K, _BLOCK_T), lambda i: (0, i)),
            pl.BlockSpec((_TOP_K, _BLOCK_T), lambda i: (0, i)),
        ],
        out_shape=[
            jax.ShapeDtypeStruct((_TOP_K, t), jnp.float32),
            jax.ShapeDtypeStruct((_TOP_K, t), jnp.int32),
        ],
    )(x, W, bias2d)
    return (wout_t.T, iout_t.T)


# speedup vs baseline: 1.6031x; 1.0022x over previous
import jax
import jax.numpy as jnp
from jax.experimental import pallas as pl

_TOP_K = 8
_BLOCK_T = 4096


def _gate_kernel(x_ref, w_ref, b_ref, wout_ref, iout_ref):
    x = x_ref[...]
    w = w_ref[...]
    logits = jax.lax.dot_general(
        w, x, (((1,), (1,)), ((), ())), preferred_element_type=jnp.float32
    )
    sig = jax.nn.sigmoid(logits)
    work = logits + b_ref[...]
    n_exp = work.shape[0]
    iota_f = jax.lax.broadcasted_iota(jnp.int32, work.shape, 0).astype(jnp.float32)
    key = iota_f + 0.5 * sig
    neg_inf = jnp.float32(-jnp.inf)
    big = jnp.float32(n_exp)
    vs = []
    for k in range(_TOP_K):
        m = jnp.max(work, axis=0, keepdims=True)
        v = jnp.min(jnp.where(work == m, key, big), axis=0, keepdims=True)
        vs.append(v)
        if k + 1 < _TOP_K:
            work = jnp.where(key == v, neg_inf, work)
    vmat = jnp.concatenate(vs, axis=0)
    idx_f = jnp.floor(vmat)
    wmat = 2.0 * (vmat - idx_f)
    wout_ref[...] = wmat / jnp.sum(wmat, axis=0, keepdims=True)
    iout_ref[...] = idx_f.astype(jnp.int32)


def kernel(x, W, expert_bias):
    t, d = x.shape
    e = W.shape[0]
    bias2d = expert_bias.reshape(e, 1)
    wout_t, iout_t = pl.pallas_call(
        _gate_kernel,
        grid=(t // _BLOCK_T,),
        in_specs=[
            pl.BlockSpec((_BLOCK_T, d), lambda i: (i, 0)),
            pl.BlockSpec((e, d), lambda i: (0, 0)),
            pl.BlockSpec((e, 1), lambda i: (0, 0)),
        ],
        out_specs=[
            pl.BlockSpec((_TOP_K, _BLOCK_T), lambda i: (0, i)),
            pl.BlockSpec((_TOP_K, _BLOCK_T), lambda i: (0, i)),
        ],
        out_shape=[
            jax.ShapeDtypeStruct((_TOP_K, t), jnp.float32),
            jax.ShapeDtypeStruct((_TOP_K, t), jnp.int32),
        ],
    )(x, W, bias2d)
    return (wout_t.T, iout_t.T)
